# baseline (speedup 1.0000x reference)
import functools

import jax
import jax.numpy as jnp
from jax import lax
from jax.experimental import pallas as pl
from jax.experimental.pallas import tpu as pltpu
from jax.experimental.pallas import tpu_sc as plsc

_INFO = plsc.get_sparse_core_info()
_NC = _INFO.num_cores
_NS = _INFO.num_subcores
_NW = _NC * _NS
_CHUNK = 128


@functools.partial(jax.jit, static_argnums=(2, 3, 4))
def _sc_embed(table, idx_flat, n_l, n_j, cpw):
    d = table.shape[1]
    dt = d // 8
    mesh = plsc.VectorSubcoreMesh(core_axis_name="c", subcore_axis_name="s")

    @functools.partial(
        pl.kernel,
        mesh=mesh,
        compiler_params=pltpu.CompilerParams(
            use_tc_tiling_on_sc=False, needs_layout_passes=False
        ),
        out_type=jax.ShapeDtypeStruct((n_l, dt, n_j, 8, 128), jnp.float32),
        scratch_types=(
            [pltpu.VMEM((cpw * _CHUNK,), jnp.int32)]
            + [pltpu.VMEM((_CHUNK, d), jnp.float32) for _ in range(2)]
            + [pltpu.VMEM((dt, 8, 128), jnp.float32) for _ in range(2)]
            + [pltpu.SemaphoreType.DMA for _ in range(4)]
        ),
    )
    def k(table_hbm, idx_hbm, out_hbm, idx_v, r0, r1, t0, t1, g0, g1, s0, s1):
        wid = lax.axis_index("s") * _NC + lax.axis_index("c")
        base_k = wid * cpw
        pltpu.sync_copy(idx_hbm.at[pl.ds(base_k * _CHUNK, cpw * _CHUNK)], idx_v)

        iota = lax.cumsum(jnp.full((16,), 1, jnp.int32)) - 1

        def gather(t, rb, gs):
            pltpu.async_copy(
                table_hbm.at[idx_v.at[pl.ds(t * _CHUNK, _CHUNK)]], rb, gs
            )

        def wait_gather(rb, gs):
            pltpu.make_async_copy(
                table_hbm.at[pl.ds(0, _CHUNK)], rb, gs
            ).wait()

        def transpose(rb, tb):
            for g in range(8):
                rvec = iota + (16 * g)
                for c in range(d):
                    val = plsc.load_gather(
                        rb, [rvec, jnp.full((16,), c, jnp.int32)]
                    )
                    tb[c // 8, c % 8, pl.ds(16 * g, 16)] = val

        def store_tiles(t, tb, ss):
            k_abs = base_k + t
            l = k_abs >> 7
            j = k_abs & 127
            for i in range(dt):
                pltpu.async_copy(tb.at[i], out_hbm.at[l, i, j], ss)

        def wait_stores(tb, ss):
            for i in range(dt):
                pltpu.make_async_copy(tb.at[i], out_hbm.at[0, i, 0], ss).wait()

        gather(0, r0, g0)
        gather(1, r1, g1)

        def body(i, carry):
            for b, (rb, tb, gs, ss) in enumerate(
                ((r0, t0, g0, s0), (r1, t1, g1, s1))
            ):
                t = i * 2 + b
                wait_gather(rb, gs)

                @pl.when(i > 0)
                def _():
                    wait_stores(tb, ss)

                transpose(rb, tb)

                @pl.when(t + 2 < cpw)
                def _():
                    gather(t + 2, rb, gs)

                store_tiles(t, tb, ss)
            return carry

        lax.fori_loop(0, cpw // 2, body, 0)
        wait_stores(t0, s0)
        wait_stores(t1, s1)

    return k(table, idx_flat)


def kernel(src_ids, embed_weight):
    b, l = src_ids.shape
    d = embed_weight.shape[1]
    n_j = b // 128
    n_blocks = l * n_j
    cpw = n_blocks // _NW
    idx = src_ids.T.reshape(-1).astype(jnp.int32)
    out5 = _sc_embed(embed_weight, idx, l, n_j, cpw)
    return out5.transpose(2, 4, 0, 1, 3).reshape(b, l, d)


# speedup vs baseline: 2.6763x; 2.6763x over previous
import functools

import jax
import jax.numpy as jnp
from jax import lax
from jax.experimental import pallas as pl
from jax.experimental.pallas import tpu as pltpu
from jax.experimental.pallas import tpu_sc as plsc

_INFO = plsc.get_sparse_core_info()
_NC = _INFO.num_cores
_NS = _INFO.num_subcores
_NW = _NC * _NS
_CHUNK = 128
_NBUF = 4
_PITCH = 129


@functools.partial(jax.jit, static_argnums=(2, 3, 4))
def _sc_embed(table, idx_flat, n_l, n_j, cpw):
    d = table.shape[1]
    dt = d // 8
    mesh = plsc.VectorSubcoreMesh(core_axis_name="c", subcore_axis_name="s")

    @functools.partial(
        pl.kernel,
        mesh=mesh,
        compiler_params=pltpu.CompilerParams(
            use_tc_tiling_on_sc=False, needs_layout_passes=False
        ),
        out_type=jax.ShapeDtypeStruct((n_l, dt, n_j, 8, 128), jnp.float32),
        scratch_types=(
            [pltpu.VMEM((cpw * _CHUNK,), jnp.int32)]
            + [pltpu.VMEM((_CHUNK, d), jnp.float32) for _ in range(_NBUF)]
            + [pltpu.VMEM((dt, 8, _PITCH), jnp.float32) for _ in range(2)]
            + [pltpu.SemaphoreType.DMA for _ in range(_NBUF + 2)]
        ),
    )
    def k(table_hbm, idx_hbm, out_hbm, idx_v, *rest):
        rbs = rest[:_NBUF]
        tbs = rest[_NBUF:_NBUF + 2]
        gss = rest[_NBUF + 2:2 * _NBUF + 2]
        sss = rest[2 * _NBUF + 2:]
        wid = lax.axis_index("s") * _NC + lax.axis_index("c")
        base_k = wid * cpw
        pltpu.sync_copy(idx_hbm.at[pl.ds(base_k * _CHUNK, cpw * _CHUNK)], idx_v)

        iota = lax.cumsum(jnp.full((16,), 1, jnp.int32)) - 1
        ones = jnp.full((16,), 1, jnp.int32)
        ivs = [(iota + 16 * h) >> 3 for h in range(d // 16)]
        svs = [(iota + 16 * h) & 7 for h in range(d // 16)]

        def gather(t, rb, gs):
            pltpu.async_copy(
                table_hbm.at[idx_v.at[pl.ds(t * _CHUNK, _CHUNK)]], rb, gs
            )

        def wait_gather(rb, gs):
            pltpu.make_async_copy(
                table_hbm.at[pl.ds(0, _CHUNK)], rb, gs
            ).wait()

        def transpose(rb, tb):
            @plsc.parallel_loop(0, _CHUNK, unroll=8)
            def _(kk):
                kv = ones * kk
                for h in range(d // 16):
                    v = rb[kk, pl.ds(16 * h, 16)]
                    plsc.store_scatter(tb, [ivs[h], svs[h], kv], v)

        def src_view(tb):
            return tb.at[pl.ds(0, dt), pl.ds(0, 8), pl.ds(0, 128)]

        def store_tiles(t, tb, ss):
            k_abs = base_k + t
            l = k_abs >> 7
            j = k_abs & 127
            pltpu.async_copy(src_view(tb), out_hbm.at[l, pl.ds(0, dt), j], ss)

        def wait_stores(tb, ss):
            pltpu.make_async_copy(
                src_view(tb), out_hbm.at[0, pl.ds(0, dt), 0], ss
            ).wait()

        for b in range(_NBUF):
            gather(b, rbs[b], gss[b])

        def body(i, carry):
            for b in range(_NBUF):
                t = i * _NBUF + b
                tb = tbs[b % 2]
                ss = sss[b % 2]
                wait_gather(rbs[b], gss[b])

                @pl.when(t >= 2)
                def _():
                    wait_stores(tb, ss)

                transpose(rbs[b], tb)

                @pl.when(t + _NBUF < cpw)
                def _():
                    gather(t + _NBUF, rbs[b], gss[b])

                store_tiles(t, tb, ss)
            return carry

        lax.fori_loop(0, cpw // _NBUF, body, 0)
        wait_stores(tbs[0], sss[0])
        wait_stores(tbs[1], sss[1])

    return k(table, idx_flat)


def kernel(src_ids, embed_weight):
    b, l = src_ids.shape
    d = embed_weight.shape[1]
    n_j = b // 128
    n_blocks = l * n_j
    cpw = n_blocks // _NW
    idx = src_ids.T.reshape(-1).astype(jnp.int32)
    out5 = _sc_embed(embed_weight, idx, l, n_j, cpw)
    return out5.transpose(2, 4, 0, 1, 3).reshape(b, l, d)
